# baseline (speedup 1.0000x reference)
import functools

import jax
import jax.numpy as jnp
from jax import lax
from jax.experimental import pallas as pl
from jax.experimental.pallas import tpu as pltpu
from jax.experimental.pallas import tpu_sc as plsc

N = 10000
E = 160000
U = 64
B = 4
SEQ = 6
DH = 128
NT = 16
CH = 64
EPT = 10368
NCH = EPT // CH
EPW = 5184
NCHX = EPW // CH
DBITS = 14
NP = 10112
RPT = NP // NT
ZR = 32
BN = 1000
NB = N // BN

f32 = jnp.float32


def _prop_body(ept, nch, per_worker, t_hbm, pk_hbm, w_hbm, out_hbm,
               acc, pk_v, w_v, rows0, rows1, rows2,
               six0, six1, six2, dix0, dix1, dix2, zbuf, gsem, ssem):
    c = lax.axis_index("c")
    s = lax.axis_index("s")

    def _z(i, carry):
        for j in range(DH // 16):
            zbuf[i, pl.ds(j * 16, 16)] = jnp.zeros((16,), f32)
        return carry

    lax.fori_loop(0, ZR, _z, 0)
    for k in range(RPT // ZR):
        pltpu.sync_copy(zbuf, acc.at[pl.ds(s * RPT + k * ZR, ZR)])
    if RPT % ZR:
        pltpu.sync_copy(zbuf.at[pl.ds(0, RPT % ZR)],
                        acc.at[pl.ds(s * RPT + (RPT // ZR) * ZR, RPT % ZR)])

    if per_worker:
        g = c * NT + s
        pk_base = g * ept
        w_base = g * ept
    else:
        pk_base = c * (NT * ept) + s * ept
        w_base = s * ept

    pltpu.sync_copy(pk_hbm.at[pl.ds(pk_base, ept)], pk_v)
    pltpu.sync_copy(w_hbm.at[pl.ds(w_base, ept)], w_v)

    plsc.subcore_barrier()

    bufs = ((rows0, six0, dix0), (rows1, six1, dix1), (rows2, six2, dix2))

    def _unpack(i, six, dix):
        for q in range(CH // 16):
            pk = pk_v[pl.ds(i * CH + q * 16, 16)]
            six[pl.ds(q * 16, 16)] = lax.shift_right_logical(pk, DBITS)
            dix[pl.ds(q * 16, 16)] = lax.bitwise_and(pk, (1 << DBITS) - 1)

    _unpack(0, six0, dix0)
    pltpu.async_copy(t_hbm.at[six0], rows0, gsem)

    def _trio(gi, carry):
        for b in range(3):
            rows, six, dix = bufs[b]
            nrows, nsix, ndix = bufs[(b + 1) % 3]
            i = gi * 3 + b

            pltpu.make_async_copy(t_hbm.at[six], rows, gsem).wait()

            @pl.when(i >= 2)
            def _drain_prev():
                pltpu.make_async_copy(nrows, acc.at[ndix], ssem).wait()

            @pl.when(i + 1 < nch)
            def _launch_next():
                _unpack(i + 1, nsix, ndix)
                pltpu.async_copy(t_hbm.at[nsix], nrows, gsem)

            def _sc16(gq, inner):
                wv = w_v[pl.ds(i * CH + gq * 16, 16)]
                base = gq * 16
                for e in range(16):
                    ws = wv[e]
                    for j in range(DH // 16):
                        sl = pl.ds(j * 16, 16)
                        rows[base + e, sl] = rows[base + e, sl] * ws
                return inner

            lax.fori_loop(0, CH // 16, _sc16, 0)

            pltpu.async_copy(rows, acc.at[dix], ssem, add=True)
        return carry

    lax.fori_loop(0, nch // 3, _trio, 0)

    for i in (nch - 2, nch - 1):
        r, _, d = bufs[i % 3]
        pltpu.make_async_copy(r, acc.at[d], ssem).wait()

    plsc.subcore_barrier()
    last = N - (NT - 1) * RPT

    @pl.when(s < NT - 1)
    def _full_writeout():
        pltpu.sync_copy(acc.at[pl.ds(s * RPT, RPT)],
                        out_hbm.at[pl.ds(c * N + s * RPT, RPT)])

    @pl.when(s == NT - 1)
    def _tail_writeout():
        pltpu.sync_copy(acc.at[pl.ds((NT - 1) * RPT, last)],
                        out_hbm.at[pl.ds(c * N + (NT - 1) * RPT, last)])


def _mk_prop(ept, nch, per_worker):
    mesh = plsc.VectorSubcoreMesh(core_axis_name="c", subcore_axis_name="s")
    return pl.kernel(
        functools.partial(_prop_body, ept, nch, per_worker),
        out_type=jax.ShapeDtypeStruct((2 * N, DH), f32),
        mesh=mesh,
        scratch_types=[
            pltpu.VMEM_SHARED((NP, DH), f32),
            pltpu.VMEM((ept,), jnp.int32),
            pltpu.VMEM((ept,), f32),
            pltpu.VMEM((CH, DH), f32),
            pltpu.VMEM((CH, DH), f32),
            pltpu.VMEM((CH, DH), f32),
            pltpu.VMEM((CH,), jnp.int32),
            pltpu.VMEM((CH,), jnp.int32),
            pltpu.VMEM((CH,), jnp.int32),
            pltpu.VMEM((CH,), jnp.int32),
            pltpu.VMEM((CH,), jnp.int32),
            pltpu.VMEM((CH,), jnp.int32),
            pltpu.VMEM((ZR, DH), f32),
            pltpu.SemaphoreType.DMA,
            pltpu.SemaphoreType.DMA,
        ],
    )


@functools.lru_cache(maxsize=None)
def _get_sc_prop():
    return _mk_prop(EPT, NCH, False)


@functools.lru_cache(maxsize=None)
def _get_sc_propx():
    return _mk_prop(EPW, NCHX, True)


def _gate_x(xg_ref, t_ref, y1_ref, p2_ref):
    xg, t, y1, p2 = xg_ref[...], t_ref[...], y1_ref[...], p2_ref[...]
    rows = []
    for bl in range(2):
        sl = slice(bl * U, (bl + 1) * U)
        rows.append(jnp.concatenate(
            [xg[:, bl * 3:(bl + 1) * 3], t[:, sl], y1[:, sl], p2[:, sl]],
            axis=1))
    return jnp.concatenate(rows, axis=0)


def _split2(v):
    return jnp.concatenate([v[:BN], v[BN:]], axis=1)


def _stack2(v):
    return jnp.concatenate([v[:, :U], v[:, U:]], axis=0)


def _ru_body(xg_ref, t_ref, y1_ref, p2_ref, w_ref, b_ref, h_ref,
             rh_ref, u_ref):
    x = _gate_x(xg_ref, t_ref, y1_ref, p2_ref)
    val = jax.nn.sigmoid(
        jnp.dot(x, w_ref[...], preferred_element_type=f32) + b_ref[...])
    h = _stack2(h_ref[...])
    rh_ref[...] = _split2(val[:, :U] * h)
    u_ref[...] = _split2(val[:, U:])


def _cu_body(xg_ref, t_ref, y1_ref, p2_ref, w_ref, b_ref, h_ref, u_ref,
             hn_ref):
    x = _gate_x(xg_ref, t_ref, y1_ref, p2_ref)
    cg = jnp.tanh(
        jnp.dot(x, w_ref[...], preferred_element_type=f32) + b_ref[...])
    u = _stack2(u_ref[...])
    hn_ref[...] = _split2(u * _stack2(h_ref[...]) + (1.0 - u) * cg)


def _dec_body(xg_ref, t_ref, y1_ref, p2_ref, w_ref, b_ref, h_ref, u_ref,
              wp_ref, bp_ref, o_ref):
    x = _gate_x(xg_ref, t_ref, y1_ref, p2_ref)
    cg = jnp.tanh(
        jnp.dot(x, w_ref[...], preferred_element_type=f32) + b_ref[...])
    u = _stack2(u_ref[...])
    hn = u * _stack2(h_ref[...]) + (1.0 - u) * cg
    o = jnp.sum(hn * wp_ref[...], axis=1, keepdims=True) + bp_ref[...]
    o_ref[...] = jnp.concatenate([o[:BN], o[BN:]], axis=1)


def _nspec(width):
    return pl.BlockSpec((BN, width), lambda c, nb: (c * NB + nb, 0))


def _wfull(shape):
    return pl.BlockSpec(shape, lambda c, nb: (0,) * len(shape))


_GRID = (2, NB)

_gmm_ru = pl.pallas_call(
    _ru_body,
    grid=_GRID,
    in_specs=[_nspec(6), _nspec(DH), _nspec(DH), _nspec(DH),
              _wfull((195, 2 * U)), _wfull((1, 2 * U)), _nspec(2 * U)],
    out_specs=[_nspec(2 * U), _nspec(2 * U)],
    out_shape=[jax.ShapeDtypeStruct((2 * N, 2 * U), f32)] * 2,
)

_gmm_cu = pl.pallas_call(
    _cu_body,
    grid=_GRID,
    in_specs=[_nspec(6), _nspec(DH), _nspec(DH), _nspec(DH),
              _wfull((195, U)), _wfull((1, U)), _nspec(2 * U),
              _nspec(2 * U)],
    out_specs=_nspec(2 * U),
    out_shape=jax.ShapeDtypeStruct((2 * N, 2 * U), f32),
)

_gmm_dec = pl.pallas_call(
    _dec_body,
    grid=_GRID,
    in_specs=[_nspec(6), _nspec(DH), _nspec(DH), _nspec(DH),
              _wfull((195, U)), _wfull((1, U)), _nspec(2 * U),
              _nspec(2 * U), _wfull((1, U)), _wfull((1, 1))],
    out_specs=_nspec(2),
    out_shape=jax.ShapeDtypeStruct((2 * N, 2), f32),
)


def _wprep(W):
    W0, W1, W2 = W[0::3], W[1::3], W[2::3]
    vx = jnp.concatenate([W0[:1] - W2[:1], W1[:1], 2.0 * W2[:1]], axis=0)
    wh = jnp.concatenate([W0[1:] - W2[1:], W1[1:], 2.0 * W2[1:]], axis=0)
    return jnp.concatenate([vx, wh], axis=0)


def kernel(inputs, edge_weight, W_ru_enc, b_ru_enc, W_c_enc, b_c_enc,
           W_ru_dec, b_ru_dec, W_c_dec, b_c_dec, W_proj, b_proj, edge_index):
    src = edge_index[0]
    dst = edge_index[1]

    ept_real = E // NT
    src16 = jnp.pad(src.reshape(NT, ept_real), ((0, 0), (0, EPT - ept_real)))
    dst16 = jnp.pad(dst.reshape(NT, ept_real), ((0, 0), (0, EPT - ept_real)))
    w16 = jnp.pad(edge_weight.reshape(NT, ept_real),
                  ((0, 0), (0, EPT - ept_real)))
    sflat = src16.reshape(-1).astype(jnp.int32)
    dflat = dst16.reshape(-1).astype(jnp.int32)
    pk16 = jnp.concatenate([(sflat << DBITS) | dflat,
                            ((sflat + N) << DBITS) | dflat])
    wflat = w16.reshape(-1)

    epw_real = E // (2 * NT)
    src32 = jnp.pad(src.reshape(2 * NT, epw_real),
                    ((0, 0), (0, EPW - epw_real)))
    dst32 = jnp.pad(dst.reshape(2 * NT, epw_real),
                    ((0, 0), (0, EPW - epw_real)))
    w32 = jnp.pad(edge_weight.reshape(2 * NT, epw_real),
                  ((0, 0), (0, EPW - epw_real)))
    pkx = ((src32.reshape(-1).astype(jnp.int32) << DBITS)
           | dst32.reshape(-1).astype(jnp.int32))
    wx = w32.reshape(-1)

    Wru_e, Wc_e = _wprep(W_ru_enc), _wprep(W_c_enc)
    Wru_d, Wc_d = _wprep(W_ru_dec), _wprep(W_c_dec)
    bru_e, bc_e = b_ru_enc.reshape(1, -1), b_c_enc.reshape(1, -1)
    bru_d, bc_d = b_ru_dec.reshape(1, -1), b_c_dec.reshape(1, -1)
    wp, bp = W_proj.reshape(1, U), b_proj.reshape(1, 1)

    sc_prop = _get_sc_prop()
    sc_propx = _get_sc_propx()

    def prop(T):
        return sc_prop(T, pk16, wflat)

    def propx(xtab):
        p = sc_propx(xtab, pkx, wx)
        return p[:N] + p[N:]

    xbt = jnp.transpose(inputs, (2, 1, 0)).reshape(N, SEQ * B)
    xtab = jnp.pad(xbt, ((0, 0), (0, DH - SEQ * B)))
    ax = propx(xtab)
    a2x = propx(ax)

    def make_xg(t):
        cols = []
        for g in (xtab, ax, a2x):
            cols.append(g[:, t * B:(t + 1) * B])
        gstk = jnp.stack(cols, axis=2)
        gstk = gstk.reshape(N, 2, 2, 3)
        return jnp.transpose(gstk, (1, 0, 2, 3)).reshape(2 * N, 6)

    zeros_t = jnp.zeros((2 * N, DH), f32)
    xg_zero = jnp.zeros((2 * N, 6), f32)

    def cell_pre(xg, h, Wru, bru, first):
        T = h
        if first:
            Y1 = P2 = zeros_t
        else:
            Y1 = prop(T)
            P2 = prop(Y1)
        rh, u = _gmm_ru(xg, T, Y1, P2, Wru, bru, h)
        Tc = rh
        if first:
            Y1c = P2c = zeros_t
        else:
            Y1c = prop(Tc)
            P2c = prop(Y1c)
        return Tc, Y1c, P2c, u

    h = jnp.zeros((2 * N, 2 * U), f32)
    for t in range(SEQ):
        xg = make_xg(t)
        Tc, Y1c, P2c, u = cell_pre(xg, h, Wru_e, bru_e, t == 0)
        h = _gmm_cu(xg, Tc, Y1c, P2c, Wc_e, bc_e, h, u)

    Tc, Y1c, P2c, u = cell_pre(xg_zero, h, Wru_d, bru_d, False)
    o = _gmm_dec(xg_zero, Tc, Y1c, P2c, Wc_d, bc_d, h, u, wp, bp)
    return o.reshape(2, N, 2).transpose(0, 2, 1).reshape(B, N)


# speedup vs baseline: 1.3738x; 1.3738x over previous
import functools

import jax
import jax.numpy as jnp
from jax import lax
from jax.experimental import pallas as pl
from jax.experimental.pallas import tpu as pltpu
from jax.experimental.pallas import tpu_sc as plsc

N = 10000
E = 160000
U = 64
B = 4
SEQ = 6
DH = 128
NT = 16
CH = 128
EPT = 10240
NCH = EPT // CH
EPW = 5120
NCHX = EPW // CH
DBITS = 14
NP = 10112
RPT = NP // NT
ZR = 32
BN = 1000
NB = N // BN

f32 = jnp.float32


def _prop_body(ept, nch, per_worker, t_hbm, pk_hbm, w_hbm, out_hbm,
               acc, pk_v, w_v, rows0, rows1,
               six0, six1, dix0, dix1, zbuf, gsem, ssem):
    c = lax.axis_index("c")
    s = lax.axis_index("s")

    def _z(i, carry):
        for j in range(DH // 16):
            zbuf[i, pl.ds(j * 16, 16)] = jnp.zeros((16,), f32)
        return carry

    lax.fori_loop(0, ZR, _z, 0)
    for k in range(RPT // ZR):
        pltpu.sync_copy(zbuf, acc.at[pl.ds(s * RPT + k * ZR, ZR)])
    if RPT % ZR:
        pltpu.sync_copy(zbuf.at[pl.ds(0, RPT % ZR)],
                        acc.at[pl.ds(s * RPT + (RPT // ZR) * ZR, RPT % ZR)])

    if per_worker:
        g = c * NT + s
        pk_base = g * ept
        w_base = g * ept
    else:
        pk_base = c * (NT * ept) + s * ept
        w_base = s * ept

    plsc.subcore_barrier()

    bufs = ((rows0, six0, dix0), (rows1, six1, dix1))
    ept2 = ept // 2
    nch2 = nch // 2

    def _unpack(i, six, dix):
        for q in range(CH // 16):
            pk = pk_v[pl.ds(i * CH + q * 16, 16)]
            six[pl.ds(q * 16, 16)] = lax.shift_right_logical(pk, DBITS)
            dix[pl.ds(q * 16, 16)] = lax.bitwise_and(pk, (1 << DBITS) - 1)

    for h in range(2):
        pltpu.sync_copy(pk_hbm.at[pl.ds(pk_base + h * ept2, ept2)], pk_v)
        pltpu.sync_copy(w_hbm.at[pl.ds(w_base + h * ept2, ept2)], w_v)

        _unpack(0, six0, dix0)
        pltpu.async_copy(t_hbm.at[six0], rows0, gsem)

        def _pair(gi, carry):
            for b in range(2):
                rows, six, dix = bufs[b]
                orows, osix, odix = bufs[1 - b]
                i = gi * 2 + b

                pltpu.make_async_copy(t_hbm.at[six], rows, gsem).wait()

                @pl.when(i >= 1)
                def _drain_prev():
                    pltpu.make_async_copy(orows, acc.at[odix], ssem).wait()

                @pl.when(i + 1 < nch2)
                def _launch_next():
                    _unpack(i + 1, osix, odix)
                    pltpu.async_copy(t_hbm.at[osix], orows, gsem)

                def _sc16(gq, inner):
                    wv = w_v[pl.ds(i * CH + gq * 16, 16)]
                    base = gq * 16
                    for e in range(16):
                        ws = wv[e]
                        for j in range(DH // 16):
                            sl = pl.ds(j * 16, 16)
                            rows[base + e, sl] = rows[base + e, sl] * ws
                    return inner

                lax.fori_loop(0, CH // 16, _sc16, 0)

                pltpu.async_copy(rows, acc.at[dix], ssem, add=True)
            return carry

        lax.fori_loop(0, nch2 // 2, _pair, 0)

    for _ in range(2):
        r, _s, d = bufs[(nch2 - 1) % 2]
        pltpu.make_async_copy(r, acc.at[d], ssem).wait()

    plsc.subcore_barrier()
    last = N - (NT - 1) * RPT

    @pl.when(s < NT - 1)
    def _full_writeout():
        pltpu.sync_copy(acc.at[pl.ds(s * RPT, RPT)],
                        out_hbm.at[pl.ds(c * N + s * RPT, RPT)])

    @pl.when(s == NT - 1)
    def _tail_writeout():
        pltpu.sync_copy(acc.at[pl.ds((NT - 1) * RPT, last)],
                        out_hbm.at[pl.ds(c * N + (NT - 1) * RPT, last)])


def _mk_prop(ept, nch, per_worker):
    mesh = plsc.VectorSubcoreMesh(core_axis_name="c", subcore_axis_name="s")
    return pl.kernel(
        functools.partial(_prop_body, ept, nch, per_worker),
        out_type=jax.ShapeDtypeStruct((2 * N, DH), f32),
        mesh=mesh,
        scratch_types=[
            pltpu.VMEM_SHARED((NP, DH), f32),
            pltpu.VMEM((ept // 2,), jnp.int32),
            pltpu.VMEM((ept // 2,), f32),
            pltpu.VMEM((CH, DH), f32),
            pltpu.VMEM((CH, DH), f32),
            pltpu.VMEM((CH,), jnp.int32),
            pltpu.VMEM((CH,), jnp.int32),
            pltpu.VMEM((CH,), jnp.int32),
            pltpu.VMEM((CH,), jnp.int32),
            pltpu.VMEM((ZR, DH), f32),
            pltpu.SemaphoreType.DMA,
            pltpu.SemaphoreType.DMA,
        ],
    )


@functools.lru_cache(maxsize=None)
def _get_sc_prop():
    return _mk_prop(EPT, NCH, False)


@functools.lru_cache(maxsize=None)
def _get_sc_propx():
    return _mk_prop(EPW, NCHX, True)


def _gate_x(xg_ref, t_ref, y1_ref, p2_ref):
    xg, t, y1, p2 = xg_ref[...], t_ref[...], y1_ref[...], p2_ref[...]
    rows = []
    for bl in range(2):
        sl = slice(bl * U, (bl + 1) * U)
        rows.append(jnp.concatenate(
            [xg[:, bl * 3:(bl + 1) * 3], t[:, sl], y1[:, sl], p2[:, sl]],
            axis=1))
    return jnp.concatenate(rows, axis=0)


def _split2(v):
    return jnp.concatenate([v[:BN], v[BN:]], axis=1)


def _stack2(v):
    return jnp.concatenate([v[:, :U], v[:, U:]], axis=0)


def _ru_body(xg_ref, t_ref, y1_ref, p2_ref, w_ref, b_ref, h_ref,
             rh_ref, u_ref):
    x = _gate_x(xg_ref, t_ref, y1_ref, p2_ref)
    val = jax.nn.sigmoid(
        jnp.dot(x, w_ref[...], preferred_element_type=f32) + b_ref[...])
    h = _stack2(h_ref[...])
    rh_ref[...] = _split2(val[:, :U] * h)
    u_ref[...] = _split2(val[:, U:])


def _cu_body(xg_ref, t_ref, y1_ref, p2_ref, w_ref, b_ref, h_ref, u_ref,
             hn_ref):
    x = _gate_x(xg_ref, t_ref, y1_ref, p2_ref)
    cg = jnp.tanh(
        jnp.dot(x, w_ref[...], preferred_element_type=f32) + b_ref[...])
    u = _stack2(u_ref[...])
    hn_ref[...] = _split2(u * _stack2(h_ref[...]) + (1.0 - u) * cg)


def _dec_body(xg_ref, t_ref, y1_ref, p2_ref, w_ref, b_ref, h_ref, u_ref,
              wp_ref, bp_ref, o_ref):
    x = _gate_x(xg_ref, t_ref, y1_ref, p2_ref)
    cg = jnp.tanh(
        jnp.dot(x, w_ref[...], preferred_element_type=f32) + b_ref[...])
    u = _stack2(u_ref[...])
    hn = u * _stack2(h_ref[...]) + (1.0 - u) * cg
    o = jnp.sum(hn * wp_ref[...], axis=1, keepdims=True) + bp_ref[...]
    o_ref[...] = jnp.concatenate([o[:BN], o[BN:]], axis=1)


def _nspec(width):
    return pl.BlockSpec((BN, width), lambda c, nb: (c * NB + nb, 0))


def _wfull(shape):
    return pl.BlockSpec(shape, lambda c, nb: (0,) * len(shape))


_GRID = (2, NB)

_gmm_ru = pl.pallas_call(
    _ru_body,
    grid=_GRID,
    in_specs=[_nspec(6), _nspec(DH), _nspec(DH), _nspec(DH),
              _wfull((195, 2 * U)), _wfull((1, 2 * U)), _nspec(2 * U)],
    out_specs=[_nspec(2 * U), _nspec(2 * U)],
    out_shape=[jax.ShapeDtypeStruct((2 * N, 2 * U), f32)] * 2,
)

_gmm_cu = pl.pallas_call(
    _cu_body,
    grid=_GRID,
    in_specs=[_nspec(6), _nspec(DH), _nspec(DH), _nspec(DH),
              _wfull((195, U)), _wfull((1, U)), _nspec(2 * U),
              _nspec(2 * U)],
    out_specs=_nspec(2 * U),
    out_shape=jax.ShapeDtypeStruct((2 * N, 2 * U), f32),
)

_gmm_dec = pl.pallas_call(
    _dec_body,
    grid=_GRID,
    in_specs=[_nspec(6), _nspec(DH), _nspec(DH), _nspec(DH),
              _wfull((195, U)), _wfull((1, U)), _nspec(2 * U),
              _nspec(2 * U), _wfull((1, U)), _wfull((1, 1))],
    out_specs=_nspec(2),
    out_shape=jax.ShapeDtypeStruct((2 * N, 2), f32),
)


def _wprep(W):
    W0, W1, W2 = W[0::3], W[1::3], W[2::3]
    vx = jnp.concatenate([W0[:1] - W2[:1], W1[:1], 2.0 * W2[:1]], axis=0)
    wh = jnp.concatenate([W0[1:] - W2[1:], W1[1:], 2.0 * W2[1:]], axis=0)
    return jnp.concatenate([vx, wh], axis=0)


def kernel(inputs, edge_weight, W_ru_enc, b_ru_enc, W_c_enc, b_c_enc,
           W_ru_dec, b_ru_dec, W_c_dec, b_c_dec, W_proj, b_proj, edge_index):
    src = edge_index[0]
    dst = edge_index[1]

    ept_real = E // NT
    src16 = jnp.pad(src.reshape(NT, ept_real), ((0, 0), (0, EPT - ept_real)))
    dst16 = jnp.pad(dst.reshape(NT, ept_real), ((0, 0), (0, EPT - ept_real)))
    w16 = jnp.pad(edge_weight.reshape(NT, ept_real),
                  ((0, 0), (0, EPT - ept_real)))
    sflat = src16.reshape(-1).astype(jnp.int32)
    dflat = dst16.reshape(-1).astype(jnp.int32)
    pk16 = jnp.concatenate([(sflat << DBITS) | dflat,
                            ((sflat + N) << DBITS) | dflat])
    wflat = w16.reshape(-1)

    epw_real = E // (2 * NT)
    src32 = jnp.pad(src.reshape(2 * NT, epw_real),
                    ((0, 0), (0, EPW - epw_real)))
    dst32 = jnp.pad(dst.reshape(2 * NT, epw_real),
                    ((0, 0), (0, EPW - epw_real)))
    w32 = jnp.pad(edge_weight.reshape(2 * NT, epw_real),
                  ((0, 0), (0, EPW - epw_real)))
    pkx = ((src32.reshape(-1).astype(jnp.int32) << DBITS)
           | dst32.reshape(-1).astype(jnp.int32))
    wx = w32.reshape(-1)

    Wru_e, Wc_e = _wprep(W_ru_enc), _wprep(W_c_enc)
    Wru_d, Wc_d = _wprep(W_ru_dec), _wprep(W_c_dec)
    bru_e, bc_e = b_ru_enc.reshape(1, -1), b_c_enc.reshape(1, -1)
    bru_d, bc_d = b_ru_dec.reshape(1, -1), b_c_dec.reshape(1, -1)
    wp, bp = W_proj.reshape(1, U), b_proj.reshape(1, 1)

    sc_prop = _get_sc_prop()
    sc_propx = _get_sc_propx()

    def prop(T):
        return sc_prop(T, pk16, wflat)

    def propx(xtab):
        p = sc_propx(xtab, pkx, wx)
        return p[:N] + p[N:]

    xbt = jnp.transpose(inputs, (2, 1, 0)).reshape(N, SEQ * B)
    xtab = jnp.pad(xbt, ((0, 0), (0, DH - SEQ * B)))
    ax = propx(xtab)
    a2x = propx(ax)

    def make_xg(t):
        cols = []
        for g in (xtab, ax, a2x):
            cols.append(g[:, t * B:(t + 1) * B])
        gstk = jnp.stack(cols, axis=2)
        gstk = gstk.reshape(N, 2, 2, 3)
        return jnp.transpose(gstk, (1, 0, 2, 3)).reshape(2 * N, 6)

    zeros_t = jnp.zeros((2 * N, DH), f32)
    xg_zero = jnp.zeros((2 * N, 6), f32)

    def cell_pre(xg, h, Wru, bru, first):
        T = h
        if first:
            Y1 = P2 = zeros_t
        else:
            Y1 = prop(T)
            P2 = prop(Y1)
        rh, u = _gmm_ru(xg, T, Y1, P2, Wru, bru, h)
        Tc = rh
        if first:
            Y1c = P2c = zeros_t
        else:
            Y1c = prop(Tc)
            P2c = prop(Y1c)
        return Tc, Y1c, P2c, u

    h = jnp.zeros((2 * N, 2 * U), f32)
    for t in range(SEQ):
        xg = make_xg(t)
        Tc, Y1c, P2c, u = cell_pre(xg, h, Wru_e, bru_e, t == 0)
        h = _gmm_cu(xg, Tc, Y1c, P2c, Wc_e, bc_e, h, u)

    Tc, Y1c, P2c, u = cell_pre(xg_zero, h, Wru_d, bru_d, False)
    o = _gmm_dec(xg_zero, Tc, Y1c, P2c, Wc_d, bc_d, h, u, wp, bp)
    return o.reshape(2, N, 2).transpose(0, 2, 1).reshape(B, N)
